# baseline (speedup 1.0000x reference)
import functools

import jax
import jax.numpy as jnp
from jax import lax
from jax.experimental import pallas as pl
from jax.experimental.pallas import tpu as pltpu
from jax.experimental.pallas import tpu_sc as plsc

B, P, C = 16, 512, 512
L = 2048
LANES = 16

NUM_CORES = 2
NUM_SUBCORES = 16
NW = NUM_CORES * NUM_SUBCORES
FRAMES_PER_TILE = (B * L) // NW
HALF = FRAMES_PER_TILE
CH = 64
NCHUNK = FRAMES_PER_TILE // CH
ZERO_ROW = B * P


def _sc_expand(table, durations):
    mesh = plsc.VectorSubcoreMesh(core_axis_name="c", subcore_axis_name="s")

    @functools.partial(
        pl.kernel,
        mesh=mesh,
        out_type=jax.ShapeDtypeStruct((B * L, C), jnp.float32),
        compiler_params=pltpu.CompilerParams(needs_layout_passes=False),
        scratch_types=[
            pltpu.VMEM((P,), jnp.int32),
            pltpu.VMEM((HALF,), jnp.int32),
            pltpu.VMEM((HALF,), jnp.int32),
            pltpu.VMEM((CH, C), jnp.float32),
            pltpu.VMEM((CH, C), jnp.float32),
            pltpu.SemaphoreType.DMA,
            pltpu.SemaphoreType.DMA,
            pltpu.SemaphoreType.DMA,
            pltpu.SemaphoreType.DMA,
        ],
    )
    def k(table_hbm, dur_hbm, out_hbm, dur_v, a_v, idx_v, buf0, buf1,
          gs0, gs1, ws0, ws1):
        wid = lax.axis_index("s") * NUM_CORES + lax.axis_index("c")
        b = wid // 2
        lo = (wid % 2) * HALF

        pltpu.sync_copy(dur_hbm.at[pl.ds(b * P, P)], dur_v)

        def zero_body(i, _):
            a_v[pl.ds(i * LANES, LANES)] = jnp.zeros((LANES,), jnp.int32)
            return 0

        lax.fori_loop(0, HALF // LANES, zero_body, 0)

        def p1_body(i, carry):
            csum_in, maxc = carry
            d = dur_v[pl.ds(i * LANES, LANES)]
            cs = plsc.cumsum(d) + csum_in
            start = cs - d
            pvec = lax.iota(jnp.int32, LANES) + i * LANES
            pos = jnp.clip(start - lo, 0, HALF - 1)
            m = (d > 0) & (start >= lo) & (start < lo + HALF)
            plsc.store_scatter(a_v, [pos], pvec, mask=m)
            before = jnp.where((d > 0) & (start < lo), pvec, 0)
            return csum_in + jnp.sum(d), jnp.maximum(maxc, jnp.max(before))

        total, maxc = lax.fori_loop(0, P // LANES, p1_body,
                                    (jnp.int32(0), jnp.int32(0)))

        def p2_body(i, mc):
            a = a_v[pl.ds(i * LANES, LANES)]
            vals = jnp.maximum(plsc.cummax(a), mc)
            t = lo + i * LANES + lax.iota(jnp.int32, LANES)
            rows = jnp.where(t < total, b * P + vals, ZERO_ROW)
            idx_v[pl.ds(i * LANES, LANES)] = rows
            return jnp.max(vals)

        lax.fori_loop(0, HALF // LANES, p2_body, maxc)

        def exp_body(i, _):
            idx_v[pl.ds(i * LANES, LANES)] = jnp.full((LANES,), ZERO_ROW, jnp.int32)
            return 0
        lax.fori_loop(0, HALF // LANES, exp_body, 0)

        bufs = (buf0, buf1)
        gsems = (gs0, gs1)
        wsems = (ws0, ws1)
        out_base = wid * FRAMES_PER_TILE
        gh = [None] * NCHUNK
        wh = [None] * NCHUNK
        for ch in range(NCHUNK):
            if ch >= 2:
                wh[ch - 2].wait()
            gh[ch] = pltpu.async_copy(
                table_hbm.at[idx_v.at[pl.ds(ch * CH, CH)]],
                bufs[ch % 2], gsems[ch % 2])
            if ch >= 1:
                gh[ch - 1].wait()
                wh[ch - 1] = pltpu.async_copy(
                    bufs[(ch - 1) % 2],
                    out_hbm.at[pl.ds(out_base + (ch - 1) * CH, CH)],
                    wsems[(ch - 1) % 2])
        gh[NCHUNK - 1].wait()
        wh[NCHUNK - 1] = pltpu.async_copy(
            bufs[(NCHUNK - 1) % 2],
            out_hbm.at[pl.ds(out_base + (NCHUNK - 1) * CH, CH)],
            wsems[(NCHUNK - 1) % 2])
        wh[NCHUNK - 2].wait()
        wh[NCHUNK - 1].wait()

    return k(table, durations)


def kernel(encoder_output, log_durations):
    ld = log_durations[..., 0]
    m = (ld > 0).astype(jnp.int32)
    durations = (jnp.floor(jnp.power(2.0, ld) + 0.0001)
                 .astype(jnp.int32) * m)
    table = jnp.concatenate(
        [encoder_output.reshape(B * P, C),
         jnp.zeros((8, C), jnp.float32)], axis=0)
    out = _sc_expand(table, durations.reshape(-1))
    return out.reshape(B, L, C)


# speedup vs baseline: 16.2903x; 16.2903x over previous
import functools

import jax
import jax.numpy as jnp
from jax import lax
from jax.experimental import pallas as pl
from jax.experimental.pallas import tpu as pltpu
from jax.experimental.pallas import tpu_sc as plsc

B, P, C = 16, 512, 512
L = 2048
LANES = 16

NUM_CORES = 2
NUM_SUBCORES = 16
NW = NUM_CORES * NUM_SUBCORES
FRAMES_PER_TILE = (B * L) // NW
HALF = FRAMES_PER_TILE
CH = 64
NCHUNK = FRAMES_PER_TILE // CH
ZPAD = 1024
ZERO_ROW = B * P


def _sc_expand(table, durations):
    mesh = plsc.VectorSubcoreMesh(core_axis_name="c", subcore_axis_name="s")

    @functools.partial(
        pl.kernel,
        mesh=mesh,
        out_type=jax.ShapeDtypeStruct((B * L, C), jnp.float32),
        compiler_params=pltpu.CompilerParams(needs_layout_passes=False),
        scratch_types=[
            pltpu.VMEM((P,), jnp.int32),
            pltpu.VMEM((HALF,), jnp.int32),
            pltpu.VMEM((HALF,), jnp.int32),
            pltpu.VMEM((CH, C), jnp.float32),
            pltpu.VMEM((CH, C), jnp.float32),
            pltpu.SemaphoreType.DMA,
            pltpu.SemaphoreType.DMA,
            pltpu.SemaphoreType.DMA,
            pltpu.SemaphoreType.DMA,
        ],
    )
    def k(table_hbm, dur_hbm, out_hbm, dur_v, a_v, idx_v, buf0, buf1,
          gs0, gs1, ws0, ws1):
        wid = lax.axis_index("s") * NUM_CORES + lax.axis_index("c")
        b = wid // 2
        lo = (wid % 2) * HALF

        pltpu.sync_copy(dur_hbm.at[pl.ds(b * P, P)], dur_v)

        def zero_body(i, _):
            a_v[pl.ds(i * LANES, LANES)] = jnp.zeros((LANES,), jnp.int32)
            return 0

        lax.fori_loop(0, HALF // LANES, zero_body, 0)

        def p1_body(i, carry):
            csum_in, maxc = carry
            d = dur_v[pl.ds(i * LANES, LANES)]
            cs = plsc.cumsum(d) + csum_in
            start = cs - d
            pvec = lax.iota(jnp.int32, LANES) + i * LANES
            pos = jnp.clip(start - lo, 0, HALF - 1)
            m = (d > 0) & (start >= lo) & (start < lo + HALF)
            plsc.store_scatter(a_v, [pos], pvec, mask=m)
            before = jnp.where((d > 0) & (start < lo), pvec, 0)
            return csum_in + jnp.sum(d), jnp.maximum(maxc, jnp.max(before))

        total, maxc = lax.fori_loop(0, P // LANES, p1_body,
                                    (jnp.int32(0), jnp.int32(0)))

        def p2_body(i, mc):
            a = a_v[pl.ds(i * LANES, LANES)]
            vals = jnp.maximum(plsc.cummax(a), mc)
            t = lo + i * LANES + lax.iota(jnp.int32, LANES)
            zrow = ZERO_ROW + ((t + wid * 37) & (ZPAD - 1))
            rows = jnp.where(t < total, b * P + vals, zrow)
            idx_v[pl.ds(i * LANES, LANES)] = rows
            return jnp.max(vals)

        lax.fori_loop(0, HALF // LANES, p2_body, maxc)

        bufs = (buf0, buf1)
        gsems = (gs0, gs1)
        wsems = (ws0, ws1)
        out_base = wid * FRAMES_PER_TILE
        gh = [None] * NCHUNK
        wh = [None] * NCHUNK
        for ch in range(NCHUNK):
            if ch >= 2:
                wh[ch - 2].wait()
            gh[ch] = pltpu.async_copy(
                table_hbm.at[idx_v.at[pl.ds(ch * CH, CH)]],
                bufs[ch % 2], gsems[ch % 2])
            if ch >= 1:
                gh[ch - 1].wait()
                wh[ch - 1] = pltpu.async_copy(
                    bufs[(ch - 1) % 2],
                    out_hbm.at[pl.ds(out_base + (ch - 1) * CH, CH)],
                    wsems[(ch - 1) % 2])
        gh[NCHUNK - 1].wait()
        wh[NCHUNK - 1] = pltpu.async_copy(
            bufs[(NCHUNK - 1) % 2],
            out_hbm.at[pl.ds(out_base + (NCHUNK - 1) * CH, CH)],
            wsems[(NCHUNK - 1) % 2])
        wh[NCHUNK - 2].wait()
        wh[NCHUNK - 1].wait()

    return k(table, durations)


def kernel(encoder_output, log_durations):
    ld = log_durations[..., 0]
    m = (ld > 0).astype(jnp.int32)
    durations = (jnp.floor(jnp.power(2.0, ld) + 0.0001)
                 .astype(jnp.int32) * m)
    table = jnp.concatenate(
        [encoder_output.reshape(B * P, C),
         jnp.zeros((ZPAD, C), jnp.float32)], axis=0)
    out = _sc_expand(table, durations.reshape(-1))
    return out.reshape(B, L, C)


# speedup vs baseline: 22.1811x; 1.3616x over previous
import functools

import jax
import jax.numpy as jnp
from jax import lax
from jax.experimental import pallas as pl
from jax.experimental.pallas import tpu as pltpu
from jax.experimental.pallas import tpu_sc as plsc

B, P, C = 16, 512, 512
L = 2048
LANES = 16

NUM_CORES = 2
NUM_SUBCORES = 16
NW = NUM_CORES * NUM_SUBCORES
FRAMES_PER_TILE = (B * L) // NW
HALF = FRAMES_PER_TILE
CH = 64
NCHUNK = FRAMES_PER_TILE // CH


def _sc_expand(table, durations):
    mesh = plsc.VectorSubcoreMesh(core_axis_name="c", subcore_axis_name="s")

    @functools.partial(
        pl.kernel,
        mesh=mesh,
        out_type=jax.ShapeDtypeStruct((B * L, C), jnp.float32),
        compiler_params=pltpu.CompilerParams(needs_layout_passes=False),
        scratch_types=[
            pltpu.VMEM((P,), jnp.int32),
            pltpu.VMEM((HALF,), jnp.int32),
            pltpu.VMEM((HALF,), jnp.int32),
            pltpu.VMEM((CH, C), jnp.float32),
            pltpu.VMEM((CH, C), jnp.float32),
            pltpu.VMEM((CH, C), jnp.float32),
            pltpu.SemaphoreType.DMA,
            pltpu.SemaphoreType.DMA,
            pltpu.SemaphoreType.DMA,
            pltpu.SemaphoreType.DMA,
        ],
    )
    def k(table_hbm, dur_hbm, out_hbm, dur_v, a_v, idx_v, buf0, buf1,
          zbuf, gs0, gs1, ws0, ws1):
        wid = lax.axis_index("s") * NUM_CORES + lax.axis_index("c")
        b = wid // 2
        lo = (wid % 2) * HALF

        pltpu.sync_copy(dur_hbm.at[pl.ds(b * P, P)], dur_v)

        def zero_body(i, _):
            a_v[pl.ds(i * LANES, LANES)] = jnp.zeros((LANES,), jnp.int32)
            return 0

        lax.fori_loop(0, HALF // LANES, zero_body, 0)

        def zbuf_body(r, _):
            for j in range(C // LANES):
                zbuf[r, pl.ds(j * LANES, LANES)] = jnp.zeros(
                    (LANES,), jnp.float32)
            return 0

        lax.fori_loop(0, CH, zbuf_body, 0)

        def p1_body(i, carry):
            csum_in, maxc = carry
            d = dur_v[pl.ds(i * LANES, LANES)]
            cs = plsc.cumsum(d) + csum_in
            start = cs - d
            pvec = lax.iota(jnp.int32, LANES) + i * LANES
            pos = jnp.clip(start - lo, 0, HALF - 1)
            m = (d > 0) & (start >= lo) & (start < lo + HALF)
            plsc.store_scatter(a_v, [pos], pvec, mask=m)
            before = jnp.where((d > 0) & (start < lo), pvec, 0)
            return csum_in + jnp.sum(d), jnp.maximum(maxc, jnp.max(before))

        total, maxc = lax.fori_loop(0, P // LANES, p1_body,
                                    (jnp.int32(0), jnp.int32(0)))
        total_rel = jnp.clip(total - lo, 0, HALF)

        def p2_body(i, mc):
            a = a_v[pl.ds(i * LANES, LANES)]
            vals = jnp.maximum(plsc.cummax(a), mc)
            t = lo + i * LANES + lax.iota(jnp.int32, LANES)
            rows = jnp.where(t < total, vals, t & (P - 1)) + b * P
            idx_v[pl.ds(i * LANES, LANES)] = rows
            return jnp.max(vals)

        lax.fori_loop(0, HALF // LANES, p2_body, maxc)

        bufs = (buf0, buf1)
        gsems = (gs0, gs1)
        wsems = (ws0, ws1)
        out_base = wid * FRAMES_PER_TILE

        def out_slice(c):
            return out_hbm.at[pl.ds(out_base + c * CH, CH)]

        def idx_slice(c):
            return table_hbm.at[idx_v.at[pl.ds(c * CH, CH)]]

        def has_real(c):
            return c * CH < total_rel

        def drain_write(c):
            pltpu.make_async_copy(zbuf, out_slice(c), wsems[c % 2]).wait()

        def start_chunk(c):
            @pl.when(has_real(c))
            def _():
                pltpu.async_copy(idx_slice(c), bufs[c % 2], gsems[c % 2])

        def finish_chunk(c):
            nreal = jnp.clip(total_rel - c * CH, 0, CH)

            @pl.when(has_real(c))
            def _():
                pltpu.make_async_copy(
                    idx_slice(c), bufs[c % 2], gsems[c % 2]).wait()

                def ztail(r, _):
                    for j in range(C // LANES):
                        bufs[c % 2][r, pl.ds(j * LANES, LANES)] = jnp.zeros(
                            (LANES,), jnp.float32)
                    return 0

                lax.fori_loop(nreal, CH, ztail, 0)
                pltpu.async_copy(bufs[c % 2], out_slice(c), wsems[c % 2])

            @pl.when(jnp.logical_not(has_real(c)))
            def _():
                pltpu.async_copy(zbuf, out_slice(c), wsems[c % 2])

        for ch in range(NCHUNK):
            if ch >= 2:
                drain_write(ch - 2)
            start_chunk(ch)
            if ch >= 1:
                finish_chunk(ch - 1)
        finish_chunk(NCHUNK - 1)
        drain_write(NCHUNK - 2)
        drain_write(NCHUNK - 1)

    return k(table, durations)


def kernel(encoder_output, log_durations):
    ld = log_durations[..., 0]
    m = (ld > 0).astype(jnp.int32)
    durations = (jnp.floor(jnp.power(2.0, ld) + 0.0001)
                 .astype(jnp.int32) * m)
    table = encoder_output.reshape(B * P, C)
    out = _sc_expand(table, durations.reshape(-1))
    return out.reshape(B, L, C)
